# baseline (speedup 1.0000x reference)
import functools

import jax
import jax.numpy as jnp
from jax import lax
from jax.experimental import pallas as pl
from jax.experimental.pallas import tpu as pltpu
from jax.experimental.pallas import tpu_sc as plsc

N = 10000
E = 320000
D = 128
EPS = 1e-5

NB = 10
BR = N // NB
NW = 32
ER = 5120
RPW = ER // NW
NPAD = 10016
RPS = NPAD // 16

_mesh = plsc.VectorSubcoreMesh(core_axis_name="c", subcore_axis_name="s")


DH = D // 2


KR = 4
NS = RPW // KR


def _sc_scatter_impl(vw1a, vw1b, packed, zeros, part, acc_sh, packed_b,
                     sx0, sx1, dx0, dx1, r0, r1, g0, g1, s0, s1):
    rows = [r0, r1]
    srcx = [sx0, sx1]
    dstx = [dx0, dx1]
    gsem = [g0, g1]
    ssem = [s0, s1]
    cid = lax.axis_index("c")
    sid = lax.axis_index("s")
    w = cid * 16 + sid
    pltpu.sync_copy(packed.at[pl.ds(w * NS, NS)], packed_b)

    def unpack(t, q):
        for k in range(KR * 8):
            v = packed_b[t, pl.ds(k * 16, 16)]
            srcx[q][pl.ds(k * 16, 16)] = jnp.bitwise_and(v, 0xFFFF)
            dstx[q][pl.ds(k * 16, 16)] = lax.shift_right_logical(v, 16)

    def gather_start(q):
        pltpu.async_copy(vw1.at[srcx[q]], rows[q], gsem[q])

    def gather_wait(q):
        pltpu.make_async_copy(vw1.at[srcx[q]], rows[q], gsem[q]).wait()

    def scatter_start(q):
        pltpu.async_copy(rows[q], acc_sh.at[dstx[q]], ssem[q], add=True)

    def scatter_wait(q):
        pltpu.make_async_copy(rows[q], acc_sh.at[dstx[q]], ssem[q]).wait()

    for half in (0, 1):
        vw1 = vw1a if half == 0 else vw1b
        pltpu.sync_copy(zeros, acc_sh.at[pl.ds(sid * RPS, RPS)])
        plsc.subcore_barrier()

        unpack(0, 0)
        gather_start(0)
        gather_wait(0)
        scatter_start(0)
        unpack(1, 1)
        gather_start(1)
        gather_wait(1)
        scatter_start(1)
        scatter_wait(0)
        unpack(2, 0)
        gather_start(0)

        def two_slots(c, carry):
            for qq in (0, 1):
                t = 2 * c + qq
                gather_wait(qq)
                scatter_start(qq)
                scatter_wait(1 - qq)
                unpack(t + 1, 1 - qq)
                gather_start(1 - qq)
            return carry

        lax.fori_loop(1, NS // 2 - 1, two_slots, 0)

        gather_wait(0)
        scatter_start(0)
        scatter_wait(1)
        unpack(NS - 1, 1)
        gather_start(1)
        gather_wait(1)
        scatter_start(1)
        scatter_wait(0)
        scatter_wait(1)

        plsc.subcore_barrier()
        pltpu.sync_copy(acc_sh.at[pl.ds(sid * RPS, RPS)], part.at[half, w])
        plsc.subcore_barrier()


_sc_scatter = pl.kernel(
    _sc_scatter_impl,
    out_type=[jax.ShapeDtypeStruct((2, NW, RPS, DH), jnp.float32)],
    mesh=_mesh,
    scratch_types=[
        pltpu.VMEM_SHARED((NPAD, DH), jnp.float32),
        pltpu.VMEM((NS, KR * D), jnp.int32),
        pltpu.VMEM((KR * D,), jnp.int32),
        pltpu.VMEM((KR * D,), jnp.int32),
        pltpu.VMEM((KR * D,), jnp.int32),
        pltpu.VMEM((KR * D,), jnp.int32),
        pltpu.VMEM((KR * D, DH), jnp.float32),
        pltpu.VMEM((KR * D, DH), jnp.float32),
        pltpu.SemaphoreType.DMA,
        pltpu.SemaphoreType.DMA,
        pltpu.SemaphoreType.DMA,
        pltpu.SemaphoreType.DMA,
    ],
    compiler_params=pltpu.CompilerParams(use_tc_tiling_on_sc=False),
)


def _sc_deg_impl(dsts3, zeros16, ones_in, degpart, deg_sh, dst_b, ones_v, sem):
    cid = lax.axis_index("c")
    sid = lax.axis_index("s")
    w = cid * 16 + sid
    pltpu.sync_copy(dsts3.at[pl.ds(w * NS, NS)], dst_b)
    pltpu.sync_copy(zeros16, deg_sh.at[pl.ds(sid * RPS, RPS)])
    pltpu.sync_copy(ones_in, ones_v)
    plsc.subcore_barrier()

    def slot(t, carry):
        pltpu.sync_copy(ones_v, deg_sh.at[dst_b.at[t]], add=True)
        return carry

    lax.fori_loop(0, NS, slot, 0)
    plsc.subcore_barrier()
    pltpu.sync_copy(deg_sh.at[pl.ds(sid * RPS, RPS)], degpart.at[w])


_sc_deg = pl.kernel(
    _sc_deg_impl,
    out_type=[jax.ShapeDtypeStruct((NW, RPS, 16), jnp.float32)],
    mesh=_mesh,
    scratch_types=[
        pltpu.VMEM_SHARED((NPAD, 16), jnp.float32),
        pltpu.VMEM((NS, KR * D), jnp.int32),
        pltpu.VMEM((KR * D, 16), jnp.float32),
        pltpu.SemaphoreType.DMA,
    ],
    compiler_params=pltpu.CompilerParams(use_tc_tiling_on_sc=False),
)


def _mm2_body(x_ref, w0_ref, b0_ref, w1_ref, b1_ref, o0_ref, o1a_ref, o1b_ref):
    x = x_ref[...]
    o0_ref[...] = jnp.dot(x, w0_ref[...], preferred_element_type=jnp.float32) + b0_ref[...]
    o1 = jnp.dot(x, w1_ref[...], preferred_element_type=jnp.float32) + b1_ref[...]
    o1a_ref[...] = o1[:, :DH]
    o1b_ref[...] = o1[:, DH:]


def _mm2(x, w0, b0, w1, b1):
    blk = pl.BlockSpec((BR, D), lambda i: (i, 0))
    hblk = pl.BlockSpec((BR, DH), lambda i: (i, 0))
    wspec = pl.BlockSpec((D, D), lambda i: (0, 0))
    bspec = pl.BlockSpec((1, D), lambda i: (0, 0))
    return pl.pallas_call(
        _mm2_body,
        grid=(NB,),
        in_specs=[blk, wspec, bspec, wspec, bspec],
        out_specs=[blk, hblk, hblk],
        out_shape=[jax.ShapeDtypeStruct((N, D), jnp.float32),
                   jax.ShapeDtypeStruct((N, DH), jnp.float32),
                   jax.ShapeDtypeStruct((N, DH), jnp.float32)],
    )(x, w0, b0.reshape(1, D), w1, b1.reshape(1, D))


def _stats_body(vw0_ref, p_ref, degp_ref, t_ref, sums_ref):
    i = pl.program_id(0)
    deg = degp_ref[0, :, 0] + degp_ref[1, :, 0]
    dinv = 1.0 / (1.0 + deg)
    agg = jnp.concatenate(
        [p_ref[0] + p_ref[1], p_ref[2] + p_ref[3]], axis=1)
    t = (vw0_ref[...] + agg) * dinv[:, None]
    t_ref[...] = t
    s = jnp.sum(t, axis=0)
    s2 = jnp.sum(t * t, axis=0)
    upd = jnp.concatenate(
        [s[None, :], s2[None, :], jnp.zeros((6, D), jnp.float32)], axis=0)

    @pl.when(i == 0)
    def _():
        sums_ref[...] = upd

    @pl.when(i > 0)
    def _():
        sums_ref[...] = sums_ref[...] + upd


def _stats(vw0, part, degpart):
    return pl.pallas_call(
        _stats_body,
        grid=(NB,),
        in_specs=[
            pl.BlockSpec((BR, D), lambda i: (i, 0)),
            pl.BlockSpec((4, BR, DH), lambda i: (0, i, 0)),
            pl.BlockSpec((2, BR, 16), lambda i: (0, i, 0)),
        ],
        out_specs=[
            pl.BlockSpec((BR, D), lambda i: (i, 0)),
            pl.BlockSpec((8, D), lambda i: (0, 0)),
        ],
        out_shape=[
            jax.ShapeDtypeStruct((N, D), jnp.float32),
            jax.ShapeDtypeStruct((8, D), jnp.float32),
        ],
    )(vw0, part, degpart)


def _apply_factory(with_res):
    def body(*refs):
        if with_res:
            t_ref, sums_ref, g_ref, be_ref, res_ref, o_ref = refs
        else:
            t_ref, sums_ref, g_ref, be_ref, o_ref = refs
        m = sums_ref[0, :] / N
        v = sums_ref[1, :] / N - m * m
        scale = g_ref[0, :] * lax.rsqrt(v + EPS)
        y = (t_ref[...] - m[None, :]) * scale[None, :] + be_ref[0, :][None, :]
        if with_res:
            y = y + res_ref[...]
        o_ref[...] = jnp.maximum(y, 0.0)

    blk = pl.BlockSpec((BR, D), lambda i: (i, 0))
    row = pl.BlockSpec((1, D), lambda i: (0, 0))
    srow = pl.BlockSpec((8, D), lambda i: (0, 0))
    in_specs = [blk, srow, row, row] + ([blk] if with_res else [])
    return pl.pallas_call(
        body,
        grid=(NB,),
        in_specs=in_specs,
        out_specs=blk,
        out_shape=jax.ShapeDtypeStruct((N, D), jnp.float32),
    )


_apply_res = _apply_factory(True)
_apply_nores = _apply_factory(False)


def kernel(features, edges, w0_0, b0_0, w1_0, b1_0, g_0, be_0,
           w0_1, b0_1, w1_1, b1_1, g_1, be_1,
           w0_2, b0_2, w1_2, b1_2, g_2, be_2):
    npad = ER * D - 2 * E
    srcs = jnp.concatenate(
        [edges[:, 1], edges[:, 0], jnp.zeros((npad,), jnp.int32)])
    dsts = jnp.concatenate(
        [edges[:, 0], edges[:, 1], jnp.full((npad,), N, jnp.int32)])
    packed = (srcs + dsts * 65536).reshape(NW * NS, KR * D)
    dsts3 = dsts.reshape(NW * NS, KR * D)
    zeros = jnp.zeros((RPS, DH), jnp.float32)
    zeros16 = jnp.zeros((RPS, 16), jnp.float32)
    ones16 = jnp.ones((KR * D, 16), jnp.float32)

    (degpart,) = _sc_deg(dsts3, zeros16, ones16)
    degpart = degpart.reshape(2, NPAD, 16)

    x = features
    params = [(w0_0, b0_0, w1_0, b1_0, g_0, be_0),
              (w0_1, b0_1, w1_1, b1_1, g_1, be_1),
              (w0_2, b0_2, w1_2, b1_2, g_2, be_2)]
    for li, (w0, b0, w1, b1, g, be) in enumerate(params):
        vw0, vw1a, vw1b = _mm2(x, w0, b0, w1, b1)
        (part,) = _sc_scatter(vw1a, vw1b, packed, zeros)
        t, sums = _stats(vw0, part.reshape(4, NPAD, DH), degpart)
        if li == 2:
            x = _apply_res(t, sums, g.reshape(1, D), be.reshape(1, D), features)
        else:
            x = _apply_nores(t, sums, g.reshape(1, D), be.reshape(1, D))
    return x
